# baseline (speedup 1.0000x reference)
import functools

import jax
import jax.numpy as jnp
import numpy as np
from jax import lax
from jax.experimental import pallas as pl
from jax.experimental.pallas import tpu as pltpu
from jax.experimental.pallas import tpu_sc as plsc

D_MODEL = 768
MAX_POSITION = 2048
LANES = 16
NUM_CORES = 2
NUM_SUBCORES = 16
NUM_WORKERS = NUM_CORES * NUM_SUBCORES
CHUNK = 32
NBUF = 3
SCALE = float(np.sqrt(np.float32(D_MODEL)))


def _positional_encoding(length, depth_full):
    depth = depth_full // 2
    positions = np.arange(0, length, dtype=np.float32)[:, None]
    depths = np.arange(depth, dtype=np.float32)[None, :] / np.float32(depth)
    angle_rates = (1.0 / (10000.0 ** depths)).astype(np.float32)
    angle_rads = positions * angle_rates
    enc = np.concatenate([np.sin(angle_rads), np.cos(angle_rads)], axis=-1)
    return jnp.asarray(enc.astype(np.float32))


def _sc_embed(table, idx, pos, batch, seq_len):
    n_rows = batch * seq_len
    ppw = seq_len // NUM_WORKERS
    pos_chunks = ppw // CHUNK
    n_chunks = pos_chunks * batch
    mesh = plsc.VectorSubcoreMesh(core_axis_name="c", subcore_axis_name="s")

    buf_types = [pltpu.VMEM((CHUNK, D_MODEL), jnp.float32)] * NBUF
    gsem_types = [pltpu.SemaphoreType.DMA] * NBUF
    wsem_types = [pltpu.SemaphoreType.DMA] * NBUF

    @functools.partial(
        pl.kernel,
        mesh=mesh,
        out_type=jax.ShapeDtypeStruct((n_rows, D_MODEL), jnp.float32),
        scratch_types=[
            pltpu.VMEM((batch * ppw,), jnp.int32),
            pltpu.VMEM((ppw, D_MODEL), jnp.float32),
        ]
        + buf_types
        + gsem_types
        + wsem_types,
    )
    def k(table_hbm, idx_hbm, pos_hbm, out_hbm, idx_v, pos_v, *rest):
        bufs = rest[:NBUF]
        gsems = rest[NBUF : 2 * NBUF]
        wsems = rest[2 * NBUF :]
        wid = lax.axis_index("s") * NUM_CORES + lax.axis_index("c")
        pbase = wid * ppw

        for b in range(batch):
            pltpu.sync_copy(
                idx_hbm.at[pl.ds(b * seq_len + pbase, ppw)],
                idx_v.at[pl.ds(b * ppw, ppw)],
            )

        def chunk_pb(c):
            return c // batch, c % batch

        def gather(c):
            p, b = chunk_pb(c)
            return pltpu.async_copy(
                table_hbm.at[idx_v.at[pl.ds(b * ppw + p * CHUNK, CHUNK)]],
                bufs[c % NBUF],
                gsems[c % NBUF],
            )

        pltpu.sync_copy(pos_hbm.at[pl.ds(pbase, ppw)], pos_v)

        LOOKAHEAD = 1
        gcopies = [None] * n_chunks
        wcopies = [None] * n_chunks
        for c in range(min(LOOKAHEAD, n_chunks)):
            gcopies[c] = gather(c)

        for c in range(n_chunks):
            nxt = c + LOOKAHEAD
            if nxt < n_chunks:
                prev_occupant = nxt - NBUF
                if prev_occupant >= 0:
                    wcopies[prev_occupant].wait()
                gcopies[nxt] = gather(nxt)

            p, b = chunk_pb(c)
            gcopies[c].wait()
            buf = bufs[c % NBUF]
            poff = p * CHUNK

            @pl.loop(0, CHUNK)
            def _row(r):
                for j in range(0, D_MODEL, LANES):
                    buf[r, pl.ds(j, LANES)] = (
                        buf[r, pl.ds(j, LANES)] * SCALE
                        + pos_v[poff + r, pl.ds(j, LANES)]
                    )

            wcopies[c] = pltpu.async_copy(
                buf,
                out_hbm.at[pl.ds(b * seq_len + pbase + p * CHUNK, CHUNK)],
                wsems[c % NBUF],
            )

        for c in range(max(0, n_chunks - NBUF + LOOKAHEAD - 1), n_chunks):
            if wcopies[c] is not None:
                wcopies[c].wait()

    return k(table, idx, pos)


def kernel(inputs, table):
    batch, seq_len = inputs.shape
    idx = jnp.reshape(inputs.astype(jnp.int32), (batch * seq_len,))
    pos = _positional_encoding(MAX_POSITION, D_MODEL)[:seq_len]
    out = _sc_embed(table, idx, pos, batch, seq_len)
    return jnp.reshape(out, (batch, seq_len, D_MODEL))


# speedup vs baseline: 1.3278x; 1.3278x over previous
import functools

import jax
import jax.numpy as jnp
import numpy as np
from jax import lax
from jax.experimental import pallas as pl
from jax.experimental.pallas import tpu as pltpu
from jax.experimental.pallas import tpu_sc as plsc

D_MODEL = 768
MAX_POSITION = 2048
LANES = 16
NUM_CORES = 2
NUM_SUBCORES = 16
NUM_WORKERS = NUM_CORES * NUM_SUBCORES
CHUNK = 32
NBUF = 3
LOOKAHEAD = 2
SCALE = float(np.sqrt(np.float32(D_MODEL)))


def _positional_encoding(length, depth_full):
    depth = depth_full // 2
    positions = np.arange(0, length, dtype=np.float32)[:, None]
    depths = np.arange(depth, dtype=np.float32)[None, :] / np.float32(depth)
    angle_rates = (1.0 / (10000.0 ** depths)).astype(np.float32)
    angle_rads = positions * angle_rates
    enc = np.concatenate([np.sin(angle_rads), np.cos(angle_rads)], axis=-1)
    return jnp.asarray(enc.astype(np.float32))


def _sc_embed(table, idx, pos, batch, seq_len):
    n_rows = batch * seq_len
    ppw = seq_len // NUM_WORKERS
    pos_chunks = ppw // CHUNK
    n_chunks = pos_chunks * batch
    mesh = plsc.VectorSubcoreMesh(core_axis_name="c", subcore_axis_name="s")

    @functools.partial(
        pl.kernel,
        mesh=mesh,
        out_type=jax.ShapeDtypeStruct((n_rows, D_MODEL), jnp.float32),
        scratch_types=[pltpu.VMEM((batch * ppw,), jnp.int32)]
        + [pltpu.VMEM((CHUNK, D_MODEL), jnp.float32)] * (pos_chunks + NBUF)
        + [pltpu.SemaphoreType.DMA] * (2 + pos_chunks + NBUF + NBUF),
    )
    def k(table_hbm, idx_hbm, pos_hbm, out_hbm, idx_v, *rest):
        pos_bufs = rest[:pos_chunks]
        bufs = rest[pos_chunks : pos_chunks + NBUF]
        isem = rest[pos_chunks + NBUF]
        psems = rest[pos_chunks + NBUF + 2 : 2 * pos_chunks + NBUF + 2]
        gsems = rest[2 * pos_chunks + NBUF + 2 : 2 * pos_chunks + 2 * NBUF + 2]
        wsems = rest[2 * pos_chunks + 2 * NBUF + 2 :]
        wid = lax.axis_index("s") * NUM_CORES + lax.axis_index("c")
        pbase = wid * ppw

        icopies = [
            pltpu.async_copy(
                idx_hbm.at[pl.ds(b * seq_len + pbase, ppw)],
                idx_v.at[pl.ds(b * ppw, ppw)],
                isem,
            )
            for b in range(batch)
        ]
        pcopies = [
            pltpu.async_copy(
                pos_hbm.at[pl.ds(pbase + p * CHUNK, CHUNK)], pos_bufs[p], psems[p]
            )
            for p in range(pos_chunks)
        ]
        for cp in icopies:
            cp.wait()

        def chunk_pb(c):
            return c // batch, c % batch

        def gather(c):
            p, b = chunk_pb(c)
            return pltpu.async_copy(
                table_hbm.at[idx_v.at[pl.ds(b * ppw + p * CHUNK, CHUNK)]],
                bufs[c % NBUF],
                gsems[c % NBUF],
            )

        gcopies = [None] * n_chunks
        wcopies = [None] * n_chunks
        for c in range(min(LOOKAHEAD, n_chunks)):
            gcopies[c] = gather(c)

        pos_waited = [False] * pos_chunks
        for c in range(n_chunks):
            nxt = c + LOOKAHEAD
            if nxt < n_chunks:
                prev_occupant = nxt - NBUF
                if prev_occupant >= 0:
                    wcopies[prev_occupant].wait()
                gcopies[nxt] = gather(nxt)

            p, b = chunk_pb(c)
            if not pos_waited[p]:
                pcopies[p].wait()
                pos_waited[p] = True
            gcopies[c].wait()
            buf = bufs[c % NBUF]
            pos_v = pos_bufs[p]

            @pl.loop(0, CHUNK)
            def _row(r):
                for j in range(0, D_MODEL, LANES):
                    buf[r, pl.ds(j, LANES)] = (
                        buf[r, pl.ds(j, LANES)] * SCALE
                        + pos_v[r, pl.ds(j, LANES)]
                    )

            wcopies[c] = pltpu.async_copy(
                buf,
                out_hbm.at[pl.ds(b * seq_len + pbase + p * CHUNK, CHUNK)],
                wsems[c % NBUF],
            )

        for c in range(max(0, n_chunks - NBUF), n_chunks):
            if wcopies[c] is not None:
                wcopies[c].wait()

    return k(table, idx, pos)


def kernel(inputs, table):
    batch, seq_len = inputs.shape
    idx = jnp.reshape(inputs.astype(jnp.int32), (batch * seq_len,))
    pos = _positional_encoding(MAX_POSITION, D_MODEL)[:seq_len]
    out = _sc_embed(table, idx, pos, batch, seq_len)
    return jnp.reshape(out, (batch, seq_len, D_MODEL))
